# Initial kernel scaffold; baseline (speedup 1.0000x reference)
import jax
import jax.numpy as jnp
from jax.experimental import pallas as pl


def kernel(x, adj, W, a_src, a_dst, bias):
    raise NotImplementedError("write your pallas kernel here")


# speedup vs baseline: 2554.4095x; 2554.4095x over previous
import jax
import jax.numpy as jnp
from jax.experimental import pallas as pl

_NEG = -1e30


def _gat_kernel(x_ref, xT_ref, adjT_ref, W_ref, Wdst_ref, WsrcT_ref, bias_ref,
                out_ref):
    x_b = x_ref[0]
    xT_b = xT_ref[0]
    maskT = adjT_ref[0] > 0.5

    h = jnp.dot(x_b, W_ref[:], preferred_element_type=jnp.float32)
    ec = jnp.dot(x_b, Wdst_ref[:], preferred_element_type=jnp.float32)
    er = jnp.dot(WsrcT_ref[:], xT_b, preferred_element_type=jnp.float32)

    num_heads = er.shape[0]
    f_per_head = W_ref.shape[1] // num_heads
    outs = []
    for hd in range(num_heads):
        e = ec[:, hd:hd + 1] + er[hd:hd + 1, :]
        e = jnp.where(e >= 0, e, 0.2 * e)
        e = jnp.where(maskT, e, _NEG)
        m = jnp.max(e, axis=1, keepdims=True)
        m = jnp.where(m <= _NEG * 0.5, 0.0, m)
        ex = jnp.exp(e - m)
        denom = jnp.sum(ex, axis=1, keepdims=True)
        alpha = ex / (denom + 1e-16)
        outs.append(jnp.dot(alpha, h[:, hd * f_per_head:(hd + 1) * f_per_head],
                            preferred_element_type=jnp.float32))
    out_ref[0] = jnp.concatenate(outs, axis=1) + bias_ref[:]


def kernel(x, adj, W, a_src, a_dst, bias):
    B, N, Din = x.shape
    H, F = a_src.shape
    HF = H * F
    eye = jnp.eye(H, dtype=W.dtype)
    Asrc = (a_src[:, :, None] * eye[:, None, :]).reshape(HF, H)
    Adst = (a_dst[:, :, None] * eye[:, None, :]).reshape(HF, H)
    Wdst = W @ Adst
    WsrcT = (W @ Asrc).T
    xT = x.transpose(0, 2, 1)
    adjT = adj.transpose(0, 2, 1)
    bias2 = bias.reshape(1, HF)

    return pl.pallas_call(
        _gat_kernel,
        grid=(B,),
        in_specs=[
            pl.BlockSpec((1, N, Din), lambda b: (b, 0, 0)),
            pl.BlockSpec((1, Din, N), lambda b: (b, 0, 0)),
            pl.BlockSpec((1, N, N), lambda b: (b, 0, 0)),
            pl.BlockSpec((Din, HF), lambda b: (0, 0)),
            pl.BlockSpec((Din, H), lambda b: (0, 0)),
            pl.BlockSpec((H, Din), lambda b: (0, 0)),
            pl.BlockSpec((1, HF), lambda b: (0, 0)),
        ],
        out_specs=pl.BlockSpec((1, N, HF), lambda b: (b, 0, 0)),
        out_shape=jax.ShapeDtypeStruct((B, N, HF), x.dtype),
    )(x, xT, adjT, W, Wdst, WsrcT, bias2)


# speedup vs baseline: 3231.1548x; 1.2649x over previous
import jax
import jax.numpy as jnp
from jax.experimental import pallas as pl
from jax.experimental.pallas import tpu as pltpu

_DNUMS_T = (((0,), (0,)), ((), ()))


def _gat_kernel(x_ref, xT_ref, adj_ref, W_ref, Wsrc_ref, WdstT_ref, bias_ref,
                out_ref):
    x_b = x_ref[0]
    xT_b = xT_ref[0]
    mask = adj_ref[0] > 0.5

    h = jnp.dot(x_b, W_ref[:], preferred_element_type=jnp.float32)
    esc = jnp.dot(x_b, Wsrc_ref[:], preferred_element_type=jnp.float32)
    edr = jnp.dot(WdstT_ref[:], xT_b, preferred_element_type=jnp.float32)

    num_heads = edr.shape[0]
    f_per_head = W_ref.shape[1] // num_heads
    ones = jnp.ones((x_b.shape[0], 1), dtype=jnp.float32)
    outs = []
    for hd in range(num_heads):
        q = esc[:, hd:hd + 1] + edr[hd:hd + 1, :]
        q = jnp.maximum(q, 0.2 * q)
        m = jnp.max(q, axis=0, keepdims=True)
        ex = jnp.exp(q - m)
        exm = jnp.where(mask, ex, 0.0)
        num = jax.lax.dot_general(exm, h[:, hd * f_per_head:(hd + 1) * f_per_head],
                                  _DNUMS_T, preferred_element_type=jnp.float32)
        den = jax.lax.dot_general(exm, ones, _DNUMS_T,
                                  preferred_element_type=jnp.float32)
        outs.append(num / (den + 1e-16))
    out_ref[0] = jnp.concatenate(outs, axis=1) + bias_ref[:]


def kernel(x, adj, W, a_src, a_dst, bias):
    B, N, Din = x.shape
    H, F = a_src.shape
    HF = H * F
    eye = jnp.eye(H, dtype=W.dtype)
    Asrc = (a_src[:, :, None] * eye[:, None, :]).reshape(HF, H)
    Adst = (a_dst[:, :, None] * eye[:, None, :]).reshape(HF, H)
    Wsrc = W @ Asrc
    WdstT = (W @ Adst).T
    xT = x.transpose(0, 2, 1)
    bias2 = bias.reshape(1, HF)

    return pl.pallas_call(
        _gat_kernel,
        grid=(B,),
        in_specs=[
            pl.BlockSpec((1, N, Din), lambda b: (b, 0, 0)),
            pl.BlockSpec((1, Din, N), lambda b: (b, 0, 0)),
            pl.BlockSpec((1, N, N), lambda b: (b, 0, 0)),
            pl.BlockSpec((Din, HF), lambda b: (0, 0)),
            pl.BlockSpec((Din, H), lambda b: (0, 0)),
            pl.BlockSpec((H, Din), lambda b: (0, 0)),
            pl.BlockSpec((1, HF), lambda b: (0, 0)),
        ],
        out_specs=pl.BlockSpec((1, N, HF), lambda b: (b, 0, 0)),
        out_shape=jax.ShapeDtypeStruct((B, N, HF), x.dtype),
        compiler_params=pltpu.CompilerParams(
            dimension_semantics=("parallel",)),
    )(x, xT, adj, W, Wsrc, WdstT, bias2)


# speedup vs baseline: 3787.2750x; 1.1721x over previous
import functools

import jax
import jax.numpy as jnp
from jax.experimental import pallas as pl
from jax.experimental.pallas import tpu as pltpu

_DNUMS_T = (((0,), (0,)), ((), ()))


def _gat_kernel(x_ref, adj_ref, W_ref, asrc_ref, adst_ref, bias_ref, out_ref,
                *, num_heads, f_per_head):
    x_b = x_ref[0]
    mask = adj_ref[0] > 0.5
    n = x_b.shape[0]
    hf = num_heads * f_per_head

    h = jnp.dot(x_b, W_ref[:], preferred_element_type=jnp.float32)

    rowg = jax.lax.broadcasted_iota(jnp.int32, (hf, num_heads), 0) // f_per_head
    colg = jax.lax.broadcasted_iota(jnp.int32, (hf, num_heads), 1)
    blk = rowg == colg
    a_src_bd = jnp.where(blk, asrc_ref[:], 0.0)
    a_dst_bd = jnp.where(blk, adst_ref[:], 0.0)

    esc = jnp.dot(h, a_src_bd, preferred_element_type=jnp.float32)
    edc = jnp.dot(h, a_dst_bd, preferred_element_type=jnp.float32)
    ident = (jax.lax.broadcasted_iota(jnp.int32, (n, n), 0)
             == jax.lax.broadcasted_iota(jnp.int32, (n, n), 1)).astype(jnp.float32)
    edr = jax.lax.dot_general(edc, ident, _DNUMS_T,
                              preferred_element_type=jnp.float32)

    ones_col = jnp.ones((n, 1), dtype=jnp.float32)
    outs = []
    for hd in range(num_heads):
        q = esc[:, hd:hd + 1] + edr[hd:hd + 1, :]
        q = jnp.maximum(q, 0.2 * q)
        m = jnp.max(q, axis=0, keepdims=True)
        ex = jnp.exp(q - m)
        exm = jnp.where(mask, ex, 0.0)
        h_aug = jnp.concatenate(
            [h[:, hd * f_per_head:(hd + 1) * f_per_head], ones_col], axis=1)
        agg = jax.lax.dot_general(exm, h_aug, _DNUMS_T,
                                  preferred_element_type=jnp.float32)
        outs.append(agg[:, :f_per_head] / (agg[:, f_per_head:] + 1e-16))
    out_ref[0] = jnp.concatenate(outs, axis=1) + bias_ref[:]


def kernel(x, adj, W, a_src, a_dst, bias):
    B, N, Din = x.shape
    H, F = a_src.shape
    HF = H * F
    asrc2 = a_src.reshape(HF, 1)
    adst2 = a_dst.reshape(HF, 1)
    bias2 = bias.reshape(1, HF)

    return pl.pallas_call(
        functools.partial(_gat_kernel, num_heads=H, f_per_head=F),
        grid=(B,),
        in_specs=[
            pl.BlockSpec((1, N, Din), lambda b: (b, 0, 0)),
            pl.BlockSpec((1, N, N), lambda b: (b, 0, 0)),
            pl.BlockSpec((Din, HF), lambda b: (0, 0)),
            pl.BlockSpec((HF, 1), lambda b: (0, 0)),
            pl.BlockSpec((HF, 1), lambda b: (0, 0)),
            pl.BlockSpec((1, HF), lambda b: (0, 0)),
        ],
        out_specs=pl.BlockSpec((1, N, HF), lambda b: (b, 0, 0)),
        out_shape=jax.ShapeDtypeStruct((B, N, HF), x.dtype),
        compiler_params=pltpu.CompilerParams(
            dimension_semantics=("parallel",)),
    )(x, adj, W, asrc2, adst2, bias2)


# speedup vs baseline: 4035.1806x; 1.0655x over previous
import functools

import jax
import jax.numpy as jnp
from jax.experimental import pallas as pl
from jax.experimental.pallas import tpu as pltpu

_DNUMS_T = (((0,), (0,)), ((), ()))


def _gat_kernel(x_ref, adj_ref, W_ref, asrc_ref, adst_ref, bias_ref, out_ref,
                *, num_heads, f_per_head):
    x_b = x_ref[0]
    mask = adj_ref[0] > 0.5
    n = x_b.shape[0]
    hf = num_heads * f_per_head

    h = jnp.dot(x_b, W_ref[:], preferred_element_type=jnp.float32)

    rowg = jax.lax.broadcasted_iota(jnp.int32, (hf, num_heads), 0) // f_per_head
    colg = jax.lax.broadcasted_iota(jnp.int32, (hf, num_heads), 1)
    blk = rowg == colg
    a_src_bd = jnp.where(blk, asrc_ref[:], 0.0)
    a_dst_bd = jnp.where(blk, adst_ref[:], 0.0)

    esc = jnp.dot(h, a_src_bd, preferred_element_type=jnp.float32)
    edr = jax.lax.dot_general(a_dst_bd, h, (((0,), (1,)), ((), ())),
                              preferred_element_type=jnp.float32)

    ones_col = jnp.ones((n, 1), dtype=jnp.float32)
    nums, dens = [], []
    for hd in range(num_heads):
        q = esc[:, hd:hd + 1] + edr[hd:hd + 1, :]
        q = jnp.maximum(q, 0.2 * q)
        m = jnp.max(q, axis=0, keepdims=True)
        ex = jnp.exp(q - m)
        exm = jnp.where(mask, ex, 0.0)
        nums.append(jax.lax.dot_general(
            exm, h[:, hd * f_per_head:(hd + 1) * f_per_head], _DNUMS_T,
            preferred_element_type=jnp.float32))
        dens.append(jax.lax.dot_general(exm, ones_col, _DNUMS_T,
                                        preferred_element_type=jnp.float32))
    recip = 1.0 / (jnp.concatenate(dens, axis=1) + 1e-16)
    outs = [nums[hd] * recip[:, hd:hd + 1] for hd in range(num_heads)]
    out_ref[0] = jnp.concatenate(outs, axis=1) + bias_ref[:]


def kernel(x, adj, W, a_src, a_dst, bias):
    B, N, Din = x.shape
    H, F = a_src.shape
    HF = H * F
    asrc2 = a_src.reshape(HF, 1)
    adst2 = a_dst.reshape(HF, 1)
    bias2 = bias.reshape(1, HF)

    return pl.pallas_call(
        functools.partial(_gat_kernel, num_heads=H, f_per_head=F),
        grid=(B,),
        in_specs=[
            pl.BlockSpec((1, N, Din), lambda b: (b, 0, 0)),
            pl.BlockSpec((1, N, N), lambda b: (b, 0, 0)),
            pl.BlockSpec((Din, HF), lambda b: (0, 0)),
            pl.BlockSpec((HF, 1), lambda b: (0, 0)),
            pl.BlockSpec((HF, 1), lambda b: (0, 0)),
            pl.BlockSpec((1, HF), lambda b: (0, 0)),
        ],
        out_specs=pl.BlockSpec((1, N, HF), lambda b: (b, 0, 0)),
        out_shape=jax.ShapeDtypeStruct((B, N, HF), x.dtype),
        compiler_params=pltpu.CompilerParams(
            dimension_semantics=("parallel",)),
    )(x, adj, W, asrc2, adst2, bias2)


# speedup vs baseline: 5727.0269x; 1.4193x over previous
import functools

import jax
import jax.numpy as jnp
from jax.experimental import pallas as pl
from jax.experimental.pallas import tpu as pltpu

_DN_LT = (((0,), (0,)), ((), ()))
_DN_RT = (((1,), (1,)), ((), ()))


def _gat_kernel(x_ref, adj_ref, W_ref, asrc_ref, adst_ref, bias_ref, out_ref,
                *, num_heads, f_per_head):
    x_b = x_ref[0]
    mask = adj_ref[0] > 0.5
    hf = num_heads * f_per_head

    h = jnp.dot(x_b, W_ref[:], preferred_element_type=jnp.float32)

    asrc_row = jnp.concatenate(
        [asrc_ref[g:g + 1, :] for g in range(num_heads)], axis=1)
    adst_row = jnp.concatenate(
        [adst_ref[g:g + 1, :] for g in range(num_heads)], axis=1)
    rowg = jax.lax.broadcasted_iota(jnp.int32, (num_heads, hf), 0)
    colg = jax.lax.broadcasted_iota(jnp.int32, (num_heads, hf), 1) // f_per_head
    blk = rowg == colg
    a_src_bdT = jnp.where(blk, asrc_row, 0.0)
    a_dst_bdT = jnp.where(blk, adst_row, 0.0)

    esc = jax.lax.dot_general(h, a_src_bdT, _DN_RT,
                              preferred_element_type=jnp.float32)
    edr = jax.lax.dot_general(a_dst_bdT, h, _DN_RT,
                              preferred_element_type=jnp.float32)

    outs = []
    for hd in range(num_heads):
        q = esc[:, hd:hd + 1] + edr[hd:hd + 1, :]
        q = jnp.maximum(q, 0.2 * q)
        m = jnp.max(q, axis=0, keepdims=True)
        ex = jnp.exp(q - m)
        exm = jnp.where(mask, ex, 0.0)
        den = jnp.sum(exm, axis=0, keepdims=True)
        alpha = exm * (1.0 / (den + 1e-16))
        outs.append(jax.lax.dot_general(
            alpha, h[:, hd * f_per_head:(hd + 1) * f_per_head], _DN_LT,
            preferred_element_type=jnp.float32))
    out_ref[0] = jnp.concatenate(outs, axis=1) + bias_ref[:]


def kernel(x, adj, W, a_src, a_dst, bias):
    B, N, Din = x.shape
    H, F = a_src.shape
    HF = H * F
    bias2 = bias.reshape(1, HF)

    return pl.pallas_call(
        functools.partial(_gat_kernel, num_heads=H, f_per_head=F),
        grid=(B,),
        in_specs=[
            pl.BlockSpec((1, N, Din), lambda b: (b, 0, 0)),
            pl.BlockSpec((1, N, N), lambda b: (b, 0, 0)),
            pl.BlockSpec((Din, HF), lambda b: (0, 0)),
            pl.BlockSpec((H, F), lambda b: (0, 0)),
            pl.BlockSpec((H, F), lambda b: (0, 0)),
            pl.BlockSpec((1, HF), lambda b: (0, 0)),
        ],
        out_specs=pl.BlockSpec((1, N, HF), lambda b: (b, 0, 0)),
        out_shape=jax.ShapeDtypeStruct((B, N, HF), x.dtype),
        compiler_params=pltpu.CompilerParams(
            dimension_semantics=("parallel",)),
    )(x, adj, W, a_src, a_dst, bias2)


# speedup vs baseline: 5943.9858x; 1.0379x over previous
import functools

import jax
import jax.numpy as jnp
from jax.experimental import pallas as pl
from jax.experimental.pallas import tpu as pltpu

_DN_LT = (((0,), (0,)), ((), ()))
_DN_RT = (((1,), (1,)), ((), ()))


def _gat_kernel(x_ref, adj_ref, W_ref, asrc_ref, adst_ref, bias_ref, out_ref,
                *, num_heads, f_per_head):
    x_b = x_ref[0]
    mask = adj_ref[0] > 0.5
    hf = num_heads * f_per_head

    h = jnp.dot(x_b, W_ref[:], preferred_element_type=jnp.float32)

    asrc_row = jnp.concatenate(
        [asrc_ref[g:g + 1, :] for g in range(num_heads)], axis=1)
    adst_row = jnp.concatenate(
        [adst_ref[g:g + 1, :] for g in range(num_heads)], axis=1)
    rowg = jax.lax.broadcasted_iota(jnp.int32, (num_heads, hf), 0)
    colg = jax.lax.broadcasted_iota(jnp.int32, (num_heads, hf), 1) // f_per_head
    blk = rowg == colg
    a_src_bdT = jnp.where(blk, asrc_row, 0.0)
    a_dst_bdT = jnp.where(blk, adst_row, 0.0)

    log2e = jnp.float32(1.4426950408889634)
    esc = jax.lax.dot_general(h, a_src_bdT, _DN_RT,
                              preferred_element_type=jnp.float32) * log2e
    edr = jax.lax.dot_general(a_dst_bdT, h, _DN_RT,
                              preferred_element_type=jnp.float32) * log2e

    outs = []
    for hd in range(num_heads):
        q = esc[:, hd:hd + 1] + edr[hd:hd + 1, :]
        q = jnp.maximum(q, 0.2 * q)
        m = jnp.max(q, axis=0, keepdims=True)
        ex = jnp.exp2(q - m)
        exm = jnp.where(mask, ex, 0.0)
        den = jnp.sum(exm, axis=0, keepdims=True)
        alpha = exm * (1.0 / (den + 1e-16))
        outs.append(jax.lax.dot_general(
            alpha, h[:, hd * f_per_head:(hd + 1) * f_per_head], _DN_LT,
            preferred_element_type=jnp.float32))
    out_ref[0] = jnp.concatenate(outs, axis=1) + bias_ref[:]


def kernel(x, adj, W, a_src, a_dst, bias):
    B, N, Din = x.shape
    H, F = a_src.shape
    HF = H * F
    bias2 = bias.reshape(1, HF)

    return pl.pallas_call(
        functools.partial(_gat_kernel, num_heads=H, f_per_head=F),
        grid=(B,),
        in_specs=[
            pl.BlockSpec((1, N, Din), lambda b: (b, 0, 0)),
            pl.BlockSpec((1, N, N), lambda b: (b, 0, 0)),
            pl.BlockSpec((Din, HF), lambda b: (0, 0)),
            pl.BlockSpec((H, F), lambda b: (0, 0)),
            pl.BlockSpec((H, F), lambda b: (0, 0)),
            pl.BlockSpec((1, HF), lambda b: (0, 0)),
        ],
        out_specs=pl.BlockSpec((1, N, HF), lambda b: (b, 0, 0)),
        out_shape=jax.ShapeDtypeStruct((B, N, HF), x.dtype),
        compiler_params=pltpu.CompilerParams(
            dimension_semantics=("parallel",)),
    )(x, adj, W, a_src, a_dst, bias2)


# speedup vs baseline: 6665.9411x; 1.1215x over previous
import functools

import jax
import jax.numpy as jnp
from jax.experimental import pallas as pl
from jax.experimental.pallas import tpu as pltpu

_DN_LT = (((0,), (0,)), ((), ()))
_DN_RT = (((1,), (1,)), ((), ()))


def _gat_kernel(x_ref, adj_ref, W_ref, asrc_ref, adst_ref, bias_ref, out_ref,
                *, num_heads, f_per_head):
    x_b = x_ref[0]
    mask = adj_ref[0] > 0.5
    hf = num_heads * f_per_head

    h = jnp.dot(x_b, W_ref[:], preferred_element_type=jnp.float32)

    asrc_row = jnp.concatenate(
        [asrc_ref[g:g + 1, :] for g in range(num_heads)], axis=1)
    adst_row = jnp.concatenate(
        [adst_ref[g:g + 1, :] for g in range(num_heads)], axis=1)
    rowg = jax.lax.broadcasted_iota(jnp.int32, (num_heads, hf), 0)
    colg = jax.lax.broadcasted_iota(jnp.int32, (num_heads, hf), 1) // f_per_head
    blk = rowg == colg
    a_src_bdT = jnp.where(blk, asrc_row, 0.0)
    a_dst_bdT = jnp.where(blk, adst_row, 0.0)

    log2e = jnp.float32(1.4426950408889634)
    esc = jax.lax.dot_general(h, a_src_bdT, _DN_RT,
                              preferred_element_type=jnp.float32) * log2e
    edr = jax.lax.dot_general(a_dst_bdT, h, _DN_RT,
                              preferred_element_type=jnp.float32) * log2e

    outs, den_rows = [], []
    for hd in range(num_heads):
        q = esc[:, hd:hd + 1] + edr[hd:hd + 1, :]
        q = jnp.maximum(q, 0.2 * q)
        m = jnp.max(q, axis=0, keepdims=True)
        ex = jnp.exp2(q - m)
        exm = jnp.where(mask, ex, 0.0)
        den_rows.append(jnp.sum(exm, axis=0, keepdims=True))
        outs.append(jax.lax.dot_general(
            exm, h[:, hd * f_per_head:(hd + 1) * f_per_head], _DN_LT,
            preferred_element_type=jnp.float32))
    dens = jnp.concatenate(den_rows, axis=0)
    ident_h = (jax.lax.broadcasted_iota(jnp.int32, (num_heads, num_heads), 0)
               == jax.lax.broadcasted_iota(jnp.int32, (num_heads, num_heads), 1)
               ).astype(jnp.float32)
    densT = jax.lax.dot_general(dens, ident_h, _DN_LT,
                                preferred_element_type=jnp.float32)
    recip = 1.0 / (densT + 1e-16)
    rep = jnp.dot(recip, blk.astype(jnp.float32),
                  preferred_element_type=jnp.float32)
    out_ref[0] = jnp.concatenate(outs, axis=1) * rep + bias_ref[:]


def kernel(x, adj, W, a_src, a_dst, bias):
    B, N, Din = x.shape
    H, F = a_src.shape
    HF = H * F
    bias2 = bias.reshape(1, HF)

    return pl.pallas_call(
        functools.partial(_gat_kernel, num_heads=H, f_per_head=F),
        grid=(B,),
        in_specs=[
            pl.BlockSpec((1, N, Din), lambda b: (b, 0, 0)),
            pl.BlockSpec((1, N, N), lambda b: (b, 0, 0)),
            pl.BlockSpec((Din, HF), lambda b: (0, 0)),
            pl.BlockSpec((H, F), lambda b: (0, 0)),
            pl.BlockSpec((H, F), lambda b: (0, 0)),
            pl.BlockSpec((1, HF), lambda b: (0, 0)),
        ],
        out_specs=pl.BlockSpec((1, N, HF), lambda b: (b, 0, 0)),
        out_shape=jax.ShapeDtypeStruct((B, N, HF), x.dtype),
        compiler_params=pltpu.CompilerParams(
            dimension_semantics=("parallel",)),
    )(x, adj, W, a_src, a_dst, bias2)
